# baseline (speedup 1.0000x reference)
import functools

import jax
import jax.numpy as jnp
from jax.experimental import pallas as pl
from jax.experimental.pallas import tpu as pltpu


def _dot_nt(a, b):
    return jax.lax.dot_general(a, b, (((1,), (1,)), ((), ())),
                               preferred_element_type=jnp.float32)


def _l1_body(g_ref, f_ref, w1t_ref, h1_ref, gb_ref, s_ref):
    @pl.when(pl.program_id(0) == 0)
    def _():
        s_ref[...] = _dot_nt(f_ref[...], w1t_ref[...])

    gblk = g_ref[...]
    m = jnp.dot(gblk, s_ref[...], preferred_element_type=jnp.float32)
    h1_ref[...] = jnp.tanh(m)
    gb_ref[...] = gblk.astype(jnp.bfloat16)


def _hilo_t(st):
    hi = st.astype(jnp.bfloat16)
    lo = (st - hi.astype(jnp.float32)).astype(jnp.bfloat16)
    return jnp.concatenate([hi, lo], axis=0)


def _mids_body(gb_ref, h1_ref, w2t_ref, w3t_ref,
               h2_ref, h3_ref,
               s2_ref, s3_ref, h2s_ref, *, p, block_m):
    i = pl.program_id(0)

    @pl.when(i == 0)
    def _():
        s2_ref[...] = _hilo_t(_dot_nt(w2t_ref[...], h1_ref[...]))

    @pl.when(i < p)
    def _():
        acc = _dot_nt(gb_ref[...], s2_ref[...])
        ko = h2_ref.shape[1]
        m2 = acc[:, :ko] + acc[:, ko:]
        h2_ref[...] = m2
        h2s_ref[pl.ds(i * block_m, block_m), :] = m2

    @pl.when(i == p)
    def _():
        s3_ref[...] = _hilo_t(_dot_nt(w3t_ref[...], h2s_ref[...]))

    @pl.when(i >= p)
    def _():
        acc = _dot_nt(gb_ref[...], s3_ref[...])
        ko = h3_ref.shape[1]
        h3_ref[...] = acc[:, :ko] + acc[:, ko:]


def _dec_body(a_ref, b_ref, h1_ref, h2_ref, adj_ref,
              h1t_ref, h2t_ref, h3t_ref, m3t_ref):
    i = pl.program_id(0)

    @pl.when(i == 0)
    def _():
        h3t_ref[...] = b_ref[...].T

    @pl.when(i == 1)
    def _():
        m3t_ref[...] = b_ref[...].T

    @pl.when(i == 2)
    def _():
        h1t_ref[...] = h1_ref[...].T

    @pl.when(i == 3)
    def _():
        h2t_ref[...] = h2_ref[...].T

    adj_ref[...] = _dot_nt(a_ref[...], b_ref[...])


def _mid_layers(gb, h1, w2t, w3t, block_m):
    n = gb.shape[0]
    d1, d2, d3 = h1.shape[1], w2t.shape[0], w3t.shape[0]
    p = n // block_m
    return pl.pallas_call(
        functools.partial(_mids_body, p=p, block_m=block_m),
        grid=(2 * p,),
        in_specs=[
            pl.BlockSpec((block_m, n),
                         lambda i: (jnp.where(i < p, i, i - p), 0)),
            pl.BlockSpec(h1.shape, lambda i: (0, 0)),
            pl.BlockSpec(w2t.shape, lambda i: (0, 0)),
            pl.BlockSpec(w3t.shape, lambda i: (0, 0)),
        ],
        out_specs=[
            pl.BlockSpec((block_m, d2), lambda i: (jnp.minimum(i, p - 1), 0)),
            pl.BlockSpec(
                (block_m, d3),
                lambda i: (jnp.where(i < p, 0, i - p), 0)),
        ],
        out_shape=[
            jax.ShapeDtypeStruct((n, d2), jnp.float32),
            jax.ShapeDtypeStruct((n, d3), jnp.float32),
        ],
        scratch_shapes=[
            pltpu.VMEM((2 * d2, n), jnp.bfloat16),
            pltpu.VMEM((2 * d3, n), jnp.bfloat16),
            pltpu.VMEM((n, d2), jnp.float32),
        ],
    )(gb, h1, w2t, w3t)


def kernel(g, f, W1, W2, W3):
    n = g.shape[0]
    d0 = f.shape[1]
    d1, d2, d3 = W1.shape[1], W2.shape[1], W3.shape[1]
    w1t, w2t, w3t = W1.T, W2.T, W3.T

    block_m = 400
    h1, gb = pl.pallas_call(
        _l1_body,
        grid=(n // block_m,),
        in_specs=[
            pl.BlockSpec((block_m, n), lambda i: (i, 0)),
            pl.BlockSpec((n, d0), lambda i: (0, 0)),
            pl.BlockSpec((d1, d0), lambda i: (0, 0)),
        ],
        out_specs=[
            pl.BlockSpec((block_m, d1), lambda i: (i, 0)),
            pl.BlockSpec((block_m, n), lambda i: (i, 0)),
        ],
        out_shape=[
            jax.ShapeDtypeStruct((n, d1), jnp.float32),
            jax.ShapeDtypeStruct((n, n), jnp.bfloat16),
        ],
        scratch_shapes=[pltpu.VMEM((n, d1), jnp.float32)],
    )(g, f, w1t)

    h2, h3 = _mid_layers(gb, h1, w2t, w3t, 1000)

    block_r = 400
    adj, h1t, h2t, h3t, m3t = pl.pallas_call(
        _dec_body,
        grid=(n // block_r,),
        in_specs=[
            pl.BlockSpec((block_r, d3), lambda i: (i, 0)),
            pl.BlockSpec((n, d3), lambda i: (0, 0)),
            pl.BlockSpec((n, d1), lambda i: (0, 0)),
            pl.BlockSpec((n, d2), lambda i: (0, 0)),
        ],
        out_specs=[
            pl.BlockSpec((block_r, n), lambda i: (i, 0)),
            pl.BlockSpec((d1, n), lambda i: (0, 0)),
            pl.BlockSpec((d2, n), lambda i: (0, 0)),
            pl.BlockSpec((d3, n), lambda i: (0, 0)),
            pl.BlockSpec((d3, n), lambda i: (0, 0)),
        ],
        out_shape=[
            jax.ShapeDtypeStruct((n, n), jnp.float32),
            jax.ShapeDtypeStruct((d1, n), jnp.float32),
            jax.ShapeDtypeStruct((d2, n), jnp.float32),
            jax.ShapeDtypeStruct((d3, n), jnp.float32),
            jax.ShapeDtypeStruct((d3, n), jnp.float32),
        ],
    )(h3, h3, h1, h2)

    return (h1t.T, h3t.T, adj, h2t.T, m3t.T)
